# Initial kernel scaffold; baseline (speedup 1.0000x reference)
import jax
import jax.numpy as jnp
from jax.experimental import pallas as pl


def kernel(x, edge_index, edge_type, W_in, b_in, W_rel, b_rel, W_self, b_self, gamma, beta):
    raise NotImplementedError("write your pallas kernel here")


# speedup vs baseline: 3.7097x; 3.7097x over previous
import functools

import jax
import jax.numpy as jnp
from jax import lax
from jax.experimental import pallas as pl
from jax.experimental.pallas import tpu as pltpu
from jax.experimental.pallas import tpu_sc as plsc

N = 10000
E = 160000
D_IN = 256
H = 512
R = 7
L = 4

NPAD = 10240
T = R + 1
CW = 128
NCB = H // CW
CB = H // 128
NC, NS = 2, 16
EPT = E // NS
EB = 80
NCHUNK = EPT // EB
NPT = NPAD // NS
BN_ROWS = 1280
NB = NPAD // BN_ROWS
EPS = 1e-5


def _row_mask(nb, rows=BN_ROWS, cols=128):
    row = nb * rows + lax.broadcasted_iota(jnp.int32, (rows, cols), 0)
    return (row < N).astype(jnp.float32)


def _proj_body(x_ref, w_ref, b_ref, o_ref):
    o_ref[...] = (
        jnp.dot(x_ref[...], w_ref[...], preferred_element_type=jnp.float32)
        + b_ref[...]
    )


def _input_proj(x, w_in, b_in):
    return pl.pallas_call(
        _proj_body,
        grid=(NB,),
        in_specs=[
            pl.BlockSpec((BN_ROWS, D_IN), lambda i: (i, 0)),
            pl.BlockSpec((D_IN, H), lambda i: (0, 0)),
            pl.BlockSpec((1, H), lambda i: (0, 0)),
        ],
        out_specs=pl.BlockSpec((BN_ROWS, H), lambda i: (i, 0)),
        out_shape=jax.ShapeDtypeStruct((NPAD, H), jnp.float32),
    )(x, w_in, b_in.reshape(1, H))


def _table_body(h_ref, w_ref, b_ref, o_ref):
    t = pl.program_id(1)
    acc = jnp.dot(h_ref[...], w_ref[0], preferred_element_type=jnp.float32)
    is_self = (t == T - 1).astype(jnp.float32)
    o_ref[0] = acc + is_self * b_ref[...]


def _rel_table(h, w_stack, bias2):
    return pl.pallas_call(
        _table_body,
        grid=(NB, T),
        in_specs=[
            pl.BlockSpec((BN_ROWS, H), lambda nb, t: (nb, 0)),
            pl.BlockSpec((1, H, H), lambda nb, t: (t, 0, 0)),
            pl.BlockSpec((1, H), lambda nb, t: (0, 0)),
        ],
        out_specs=pl.BlockSpec((1, BN_ROWS, H), lambda nb, t: (t, nb, 0)),
        out_shape=jax.ShapeDtypeStruct((T, NPAD, H), jnp.float32),
    )(h, w_stack, bias2)


def _sc_msgs_body(table, idxs, dsts, out, idx_buf, dst_buf, rows, acc, sem):
    sc = lax.axis_index("c")
    tid = lax.axis_index("s")
    pltpu.sync_copy(idxs.at[tid], idx_buf)
    pltpu.sync_copy(dsts.at[tid], dst_buf)
    r0 = tid * NPT
    for j2 in range(NCB // NC):
        c = sc * (NCB // NC) + j2
        pltpu.sync_copy(
            table.at[pl.ds(R * NPAD + r0, NPT), pl.ds(c * CW, CW)],
            acc.at[pl.ds(r0, NPT)],
        )
        plsc.subcore_barrier()

        def body(j, _):
            pltpu.async_copy(
                table.at[idx_buf.at[j], pl.ds(c * CW, CW)], rows, sem
            ).wait()
            pltpu.sync_copy(rows, acc.at[dst_buf.at[j]], add=True)
            return ()

        lax.fori_loop(0, NCHUNK, body, ())
        plsc.subcore_barrier()
        pltpu.sync_copy(acc.at[pl.ds(r0, NPT)], out.at[pl.ds(c * NPAD + r0, NPT)])
        plsc.subcore_barrier()


_sc_msgs = functools.partial(
    pl.kernel,
    _sc_msgs_body,
    out_type=jax.ShapeDtypeStruct((NCB * NPAD, CW), jnp.float32),
    mesh=plsc.VectorSubcoreMesh(
        core_axis_name="c", subcore_axis_name="s", num_cores=NC, num_subcores=NS
    ),
    scratch_types=[
        pltpu.VMEM((NCHUNK, EB), jnp.int32),
        pltpu.VMEM((NCHUNK, EB), jnp.int32),
        pltpu.VMEM((EB, CW), jnp.float32),
        pltpu.VMEM_SHARED((NPAD, CW), jnp.float32),
        pltpu.SemaphoreType.DMA,
    ],
)()


def _bn_body(m_ref, h_ref, g_ref, b_ref, o_ref, stats):
    p = pl.program_id(0)
    c = pl.program_id(1)
    nb = pl.program_id(2)
    blk = m_ref[0]

    @pl.when(p == 0)
    def _():
        m = _row_mask(nb)
        mblk = blk * m
        s = jnp.sum(mblk, axis=0, keepdims=True)
        s2 = jnp.sum(mblk * blk, axis=0, keepdims=True)

        @pl.when(nb == 0)
        def _():
            stats[0, c] = s[0]
            stats[1, c] = s2[0]

        @pl.when(nb != 0)
        def _():
            stats[0, c] += s[0]
            stats[1, c] += s2[0]

    @pl.when(p == 1)
    def _():
        mean = stats[0, c] * (1.0 / N)
        var = stats[1, c] * (1.0 / N) - mean * mean
        inv = lax.rsqrt(var + EPS) * g_ref[0, 0]
        out = (blk - mean) * inv + b_ref[0, 0]
        o_ref[...] = jnp.maximum(out, 0.0) + h_ref[...]


def _bn_apply(msg, h, gamma2, beta2):
    return pl.pallas_call(
        _bn_body,
        grid=(2, CB, NB),
        in_specs=[
            pl.BlockSpec((1, BN_ROWS, CW), lambda p, c, nb: (c, nb, 0)),
            pl.BlockSpec((BN_ROWS, 128), lambda p, c, nb: (nb, c)),
            pl.BlockSpec((1, 1, 128), lambda p, c, nb: (c, 0, 0)),
            pl.BlockSpec((1, 1, 128), lambda p, c, nb: (c, 0, 0)),
        ],
        out_specs=pl.BlockSpec((BN_ROWS, 128), lambda p, c, nb: (nb, c)),
        out_shape=jax.ShapeDtypeStruct((NPAD, H), jnp.float32),
        scratch_shapes=[pltpu.VMEM((2, CB, 128), jnp.float32)],
    )(msg, h, gamma2, beta2)


def _mean_body(h_ref, o_ref):
    nb = pl.program_id(0)
    m = _row_mask(nb, BN_ROWS, H)
    s = jnp.sum(h_ref[...] * m, axis=0, keepdims=True)

    @pl.when(nb == 0)
    def _():
        o_ref[...] = s

    @pl.when(nb != 0)
    def _():
        o_ref[...] += s

    @pl.when(nb == NB - 1)
    def _():
        o_ref[...] *= 1.0 / N


def _graph_mean(h):
    return pl.pallas_call(
        _mean_body,
        grid=(NB,),
        in_specs=[pl.BlockSpec((BN_ROWS, H), lambda nb: (nb, 0))],
        out_specs=pl.BlockSpec((1, H), lambda nb: (0, 0)),
        out_shape=jax.ShapeDtypeStruct((1, H), jnp.float32),
    )(h)


def kernel(x, edge_index, edge_type, W_in, b_in, W_rel, b_rel, W_self, b_self, gamma, beta):
    src = edge_index[0]
    dst = edge_index[1]
    idxs = (edge_type * NPAD + src).reshape(NS, NCHUNK, EB)
    dst3 = dst.reshape(NS, NCHUNK, EB)

    w_stack = jnp.concatenate([W_rel.reshape(L, R, H, H), W_self[:, None]], axis=1)
    bias2 = (b_rel + b_self).reshape(L, 1, H)
    gamma2 = gamma.reshape(L, CB, 1, 128)
    beta2 = beta.reshape(L, CB, 1, 128)

    x_pad = jnp.pad(x, ((0, NPAD - N), (0, 0)))
    h = _input_proj(x_pad, W_in, b_in)
    for l in range(L):
        table = _rel_table(h, w_stack[l], bias2[l]).reshape(T * NPAD, H)
        msg = _sc_msgs(table, idxs, dst3).reshape(NCB, NPAD, CW)
        h = _bn_apply(msg, h, gamma2[l], beta2[l])
    graph = _graph_mean(h)[0]
    return h[:N], graph
